# baseline (speedup 1.0000x reference)
import functools

import jax
import jax.numpy as jnp
from jax import lax
from jax.experimental import pallas as pl
from jax.experimental.pallas import tpu as pltpu
from jax.experimental.pallas import tpu_sc as plsc

NV = 2048
N_CLAUSES = 8192
BATCH = 1024

_NUM_CORES = 2
_NUM_SUBCORES = 16
_NUM_WORKERS = _NUM_CORES * _NUM_SUBCORES
_CPW = N_CLAUSES // _NUM_WORKERS
_LANES = 16


def _sc_body(t_hbm, e0_hbm, e1_hbm, e2_hbm,
             out_hbm, t_v, e0_v, e1_v, e2_v, res_v, sem):
    wid = lax.axis_index("s") * _NUM_CORES + lax.axis_index("c")
    base = wid * _CPW
    sl_in = pl.ds(base, _CPW)
    copies = [
        pltpu.async_copy(t_hbm, t_v, sem),
        pltpu.async_copy(e0_hbm.at[sl_in], e0_v, sem),
        pltpu.async_copy(e1_hbm.at[sl_in], e1_v, sem),
        pltpu.async_copy(e2_hbm.at[sl_in], e2_v, sem),
    ]
    for c in copies:
        c.wait()
    @plsc.parallel_loop(0, _CPW, _LANES, unroll=8)
    def _(off):
        sl = pl.ds(off, _LANES)
        vals = []
        for ev_ref in (e0_v, e1_v, e2_v):
            ev = ev_ref[sl]
            tv = plsc.load_gather(t_v, [lax.shift_right_logical(ev, 1)])
            sv = jnp.where(lax.eq(lax.bitwise_and(ev, 1), 1), 1.0, -1.0)
            vals.append(1.0 / (1.0 + jnp.exp(tv * sv)))
        res_v[sl] = 1.0 - vals[0] * vals[1] * vals[2]
    pltpu.sync_copy(res_v, out_hbm.at[pl.ds(base, _CPW)])


_sc_clauses = functools.partial(
    pl.kernel,
    out_type=jax.ShapeDtypeStruct((N_CLAUSES,), jnp.float32),
    mesh=plsc.VectorSubcoreMesh(core_axis_name="c", subcore_axis_name="s"),
    compiler_params=pltpu.CompilerParams(needs_layout_passes=False),
    scratch_types=[
        pltpu.VMEM((NV,), jnp.float32),
        pltpu.VMEM((_CPW,), jnp.int32),
        pltpu.VMEM((_CPW,), jnp.int32),
        pltpu.VMEM((_CPW,), jnp.int32),
        pltpu.VMEM((_CPW,), jnp.float32),
        pltpu.SemaphoreType.DMA,
    ],
)(_sc_body)


_REP_ROWS = 128


def _sc_full_body(t_hbm, i0_hbm, i1_hbm, i2_hbm, s0_hbm, s1_hbm, s2_hbm,
                  out_hbm, t_v, i0_v, i1_v, i2_v, s0_v, s1_v, s2_v, res_v,
                  rep_v, sem, osem):
    wid = lax.axis_index("s") * _NUM_CORES + lax.axis_index("c")
    base = wid * _CPW
    sl_in = pl.ds(base, _CPW)
    copies = [
        pltpu.async_copy(t_hbm, t_v, sem),
        pltpu.async_copy(i0_hbm.at[sl_in], i0_v, sem),
        pltpu.async_copy(i1_hbm.at[sl_in], i1_v, sem),
        pltpu.async_copy(i2_hbm.at[sl_in], i2_v, sem),
        pltpu.async_copy(s0_hbm.at[sl_in], s0_v, sem),
        pltpu.async_copy(s1_hbm.at[sl_in], s1_v, sem),
        pltpu.async_copy(s2_hbm.at[sl_in], s2_v, sem),
    ]
    for c in copies:
        c.wait()
    for j in range(_CPW // _LANES):
        sl = pl.ds(j * _LANES, _LANES)
        vals = []
        for iv_ref, sv_ref in ((i0_v, s0_v), (i1_v, s1_v), (i2_v, s2_v)):
            tv = plsc.load_gather(t_v, [iv_ref[sl]])
            sv = sv_ref[sl].astype(jnp.float32)
            vals.append(1.0 / (1.0 + jnp.exp(tv * sv)))
        res_v[sl] = 1.0 - vals[0] * vals[1] * vals[2]
    for r in range(_REP_ROWS):
        for j in range(_CPW // _LANES):
            sl = pl.ds(j * _LANES, _LANES)
            rep_v[r, sl] = res_v[sl]
    outs = [
        pltpu.async_copy(
            rep_v, out_hbm.at[pl.ds(b * _REP_ROWS, _REP_ROWS), sl_in], osem)
        for b in range(BATCH // _REP_ROWS)
    ]
    for c in outs:
        c.wait()


_sc_full = functools.partial(
    pl.kernel,
    out_type=jax.ShapeDtypeStruct((BATCH, N_CLAUSES), jnp.float32),
    mesh=plsc.VectorSubcoreMesh(core_axis_name="c", subcore_axis_name="s"),
    compiler_params=pltpu.CompilerParams(needs_layout_passes=False),
    scratch_types=[
        pltpu.VMEM((NV,), jnp.float32),
        pltpu.VMEM((_CPW,), jnp.int32),
        pltpu.VMEM((_CPW,), jnp.int32),
        pltpu.VMEM((_CPW,), jnp.int32),
        pltpu.VMEM((_CPW,), jnp.int32),
        pltpu.VMEM((_CPW,), jnp.int32),
        pltpu.VMEM((_CPW,), jnp.int32),
        pltpu.VMEM((_CPW,), jnp.float32),
        pltpu.VMEM((_REP_ROWS, _CPW), jnp.float32),
        pltpu.SemaphoreType.DMA,
        pltpu.SemaphoreType.DMA,
    ],
)(_sc_full_body)


def _bcast_body(vec_ref, out_ref):
    out_ref[...] = jnp.broadcast_to(vec_ref[...], out_ref.shape)


_ROWS_PER_BLOCK = 128


def _broadcast(vec):
    return pl.pallas_call(
        _bcast_body,
        grid=(BATCH // _ROWS_PER_BLOCK,),
        in_specs=[pl.BlockSpec((1, N_CLAUSES), lambda i: (0, 0))],
        out_specs=pl.BlockSpec((_ROWS_PER_BLOCK, N_CLAUSES), lambda i: (i, 0)),
        out_shape=jax.ShapeDtypeStruct((BATCH, N_CLAUSES), jnp.float32),
    )(vec.reshape(1, N_CLAUSES))


def kernel(input, emb_weight, clause_idx, clause_sign):
    del input
    t = emb_weight.reshape(NV)
    e = clause_idx * 2 + (clause_sign > 0).astype(jnp.int32)
    vec = _sc_clauses(t, e[:, 0], e[:, 1], e[:, 2])
    return _broadcast(vec)


# speedup vs baseline: 1.0621x; 1.0621x over previous
import functools

import jax
import jax.numpy as jnp
from jax import lax
from jax.experimental import pallas as pl
from jax.experimental.pallas import tpu as pltpu
from jax.experimental.pallas import tpu_sc as plsc

NV = 2048
N_CLAUSES = 8192
BATCH = 1024

_NUM_CORES = 1
_NUM_SUBCORES = 16
_NUM_WORKERS = _NUM_CORES * _NUM_SUBCORES
_CPW = N_CLAUSES // _NUM_WORKERS
_LANES = 16


def _sc_body(t_hbm, e0_hbm, e1_hbm, e2_hbm,
             out_hbm, t_v, e0_v, e1_v, e2_v, res_v, sem):
    wid = lax.axis_index("s") * _NUM_CORES + lax.axis_index("c")
    base = wid * _CPW
    sl_in = pl.ds(base, _CPW)
    copies = [
        pltpu.async_copy(t_hbm, t_v, sem),
        pltpu.async_copy(e0_hbm.at[sl_in], e0_v, sem),
        pltpu.async_copy(e1_hbm.at[sl_in], e1_v, sem),
        pltpu.async_copy(e2_hbm.at[sl_in], e2_v, sem),
    ]
    for c in copies:
        c.wait()
    @plsc.parallel_loop(0, _CPW, _LANES, unroll=4)
    def _(off):
        sl = pl.ds(off, _LANES)
        vals = []
        for ev_ref in (e0_v, e1_v, e2_v):
            ev = ev_ref[sl]
            tv = plsc.load_gather(t_v, [lax.shift_right_logical(ev, 1)])
            sv = jnp.where(lax.eq(lax.bitwise_and(ev, 1), 1), 1.0, -1.0)
            vals.append(1.0 / (1.0 + jnp.exp(tv * sv)))
        res_v[sl] = 1.0 - vals[0] * vals[1] * vals[2]
    pltpu.sync_copy(res_v, out_hbm.at[pl.ds(base, _CPW)])


_sc_clauses = functools.partial(
    pl.kernel,
    out_type=jax.ShapeDtypeStruct((N_CLAUSES,), jnp.float32),
    mesh=plsc.VectorSubcoreMesh(
        core_axis_name="c", subcore_axis_name="s", num_cores=_NUM_CORES),
    compiler_params=pltpu.CompilerParams(needs_layout_passes=False),
    scratch_types=[
        pltpu.VMEM((NV,), jnp.float32),
        pltpu.VMEM((_CPW,), jnp.int32),
        pltpu.VMEM((_CPW,), jnp.int32),
        pltpu.VMEM((_CPW,), jnp.int32),
        pltpu.VMEM((_CPW,), jnp.float32),
        pltpu.SemaphoreType.DMA,
    ],
)(_sc_body)


_REP_ROWS = 128


def _sc_full_body(t_hbm, i0_hbm, i1_hbm, i2_hbm, s0_hbm, s1_hbm, s2_hbm,
                  out_hbm, t_v, i0_v, i1_v, i2_v, s0_v, s1_v, s2_v, res_v,
                  rep_v, sem, osem):
    wid = lax.axis_index("s") * _NUM_CORES + lax.axis_index("c")
    base = wid * _CPW
    sl_in = pl.ds(base, _CPW)
    copies = [
        pltpu.async_copy(t_hbm, t_v, sem),
        pltpu.async_copy(i0_hbm.at[sl_in], i0_v, sem),
        pltpu.async_copy(i1_hbm.at[sl_in], i1_v, sem),
        pltpu.async_copy(i2_hbm.at[sl_in], i2_v, sem),
        pltpu.async_copy(s0_hbm.at[sl_in], s0_v, sem),
        pltpu.async_copy(s1_hbm.at[sl_in], s1_v, sem),
        pltpu.async_copy(s2_hbm.at[sl_in], s2_v, sem),
    ]
    for c in copies:
        c.wait()
    for j in range(_CPW // _LANES):
        sl = pl.ds(j * _LANES, _LANES)
        vals = []
        for iv_ref, sv_ref in ((i0_v, s0_v), (i1_v, s1_v), (i2_v, s2_v)):
            tv = plsc.load_gather(t_v, [iv_ref[sl]])
            sv = sv_ref[sl].astype(jnp.float32)
            vals.append(1.0 / (1.0 + jnp.exp(tv * sv)))
        res_v[sl] = 1.0 - vals[0] * vals[1] * vals[2]
    for r in range(_REP_ROWS):
        for j in range(_CPW // _LANES):
            sl = pl.ds(j * _LANES, _LANES)
            rep_v[r, sl] = res_v[sl]
    outs = [
        pltpu.async_copy(
            rep_v, out_hbm.at[pl.ds(b * _REP_ROWS, _REP_ROWS), sl_in], osem)
        for b in range(BATCH // _REP_ROWS)
    ]
    for c in outs:
        c.wait()


_sc_full = functools.partial(
    pl.kernel,
    out_type=jax.ShapeDtypeStruct((BATCH, N_CLAUSES), jnp.float32),
    mesh=plsc.VectorSubcoreMesh(core_axis_name="c", subcore_axis_name="s"),
    compiler_params=pltpu.CompilerParams(needs_layout_passes=False),
    scratch_types=[
        pltpu.VMEM((NV,), jnp.float32),
        pltpu.VMEM((_CPW,), jnp.int32),
        pltpu.VMEM((_CPW,), jnp.int32),
        pltpu.VMEM((_CPW,), jnp.int32),
        pltpu.VMEM((_CPW,), jnp.int32),
        pltpu.VMEM((_CPW,), jnp.int32),
        pltpu.VMEM((_CPW,), jnp.int32),
        pltpu.VMEM((_CPW,), jnp.float32),
        pltpu.VMEM((_REP_ROWS, _CPW), jnp.float32),
        pltpu.SemaphoreType.DMA,
        pltpu.SemaphoreType.DMA,
    ],
)(_sc_full_body)


def _bcast_body(vec_ref, out_ref):
    out_ref[...] = jnp.broadcast_to(vec_ref[...], out_ref.shape)


_ROWS_PER_BLOCK = 128


def _broadcast(vec):
    return pl.pallas_call(
        _bcast_body,
        grid=(BATCH // _ROWS_PER_BLOCK,),
        in_specs=[pl.BlockSpec((1, N_CLAUSES), lambda i: (0, 0))],
        out_specs=pl.BlockSpec((_ROWS_PER_BLOCK, N_CLAUSES), lambda i: (i, 0)),
        out_shape=jax.ShapeDtypeStruct((BATCH, N_CLAUSES), jnp.float32),
    )(vec.reshape(1, N_CLAUSES))


def kernel(input, emb_weight, clause_idx, clause_sign):
    del input
    t = emb_weight.reshape(NV)
    e = clause_idx * 2 + (clause_sign > 0).astype(jnp.int32)
    vec = _sc_clauses(t, e[:, 0], e[:, 1], e[:, 2])
    return _broadcast(vec)
